# baseline (speedup 1.0000x reference)
import jax
import jax.numpy as jnp
from jax import lax
from jax.experimental import pallas as pl
from jax.experimental.pallas import tpu as pltpu
from jax.experimental.pallas import tpu_sc as plsc

B = 4096
L = 50
D = 64
NCLS = 6
V = 1000000
C8 = 8

NC = 2
NS = 16
NW = NC * NS
B_PER_W = B // NW
CHUNK = 128
NCHUNK = (B_PER_W * L) // CHUNK
ROWS_PER_SC = B // NC

BN = 65536


def _tw_body(wp_ref, tt_ref, *plane_refs):
    res = lax.dot_general(
        wp_ref[...].astype(jnp.bfloat16), tt_ref[...].astype(jnp.bfloat16),
        dimension_numbers=(((1,), (0,)), ((), ())),
        preferred_element_type=jnp.float32,
    ) * (1.0 / L)
    for p in range(NP):
        lo = lax.bitcast_convert_type(
            res[2 * p, :].astype(jnp.bfloat16), jnp.uint16).astype(jnp.uint32)
        hi = lax.bitcast_convert_type(
            res[2 * p + 1, :].astype(jnp.bfloat16), jnp.uint16).astype(jnp.uint32)
        plane_refs[p][...] = lax.bitcast_convert_type(
            (hi << 16) | lo, jnp.float32)


def _tw_planes(wp, table_t):
    return pl.pallas_call(
        _tw_body,
        grid=(pl.cdiv(V, BN),),
        in_specs=[
            pl.BlockSpec((C8, D), lambda i: (0, 0)),
            pl.BlockSpec((D, BN), lambda i: (0, i)),
        ],
        out_specs=[pl.BlockSpec((BN,), lambda i: (i,))] * NP,
        out_shape=[jax.ShapeDtypeStruct((V,), jnp.float32)] * NP,
    )(wp, table_t)


NP = NCLS // 2
NBUF = 8
LANES = 16
VPC = CHUNK // LANES


def _sc_pool_body(x_hbm, pat_hbm, binit_hbm,
                  p0, p1, p2,
                  out_hbm, idxs, pat, rows, acc, gsem):
    planes = (p0, p1, p2)
    c = lax.axis_index("c")
    s = lax.axis_index("s")
    wid = c * NS + s

    pltpu.sync_copy(x_hbm.at[wid], idxs)
    pltpu.sync_copy(pat_hbm, pat)
    pltpu.sync_copy(binit_hbm, acc)

    def _gather(g, buf):
        for k in range(NP):
            pltpu.async_copy(planes[k].at[idxs.at[g]],
                             rows.at[pl.ds((buf * NP + k) * CHUNK, CHUNK)],
                             gsem)

    def _wait(g, buf):
        for k in range(NP):
            pltpu.make_async_copy(
                planes[k].at[idxs.at[g]],
                rows.at[pl.ds((buf * NP + k) * CHUNK, CHUNK)],
                gsem).wait()

    for g in range(NBUF):
        _gather(g, g)

    def body(g, _):
        buf = lax.rem(g, NBUF)
        _wait(g, buf)
        for k in range(NP):
            for j in range(VPC):
                val = rows[pl.ds((buf * NP + k) * CHUNK + j * LANES,
                                 LANES)]
                u = lax.bitcast_convert_type(val, jnp.uint32)
                lo = lax.bitcast_convert_type(u << 16, jnp.float32)
                hi = lax.bitcast_convert_type(
                    u & jnp.uint32(0xFFFF0000), jnp.float32)
                plsc.addupdate_scatter(
                    acc, [pat[(2 * k) * VPC + j, :]], lo)
                plsc.addupdate_scatter(
                    acc, [pat[(2 * k + 1) * VPC + j, :]], hi)
        nxt = g + NBUF

        @pl.when(nxt < NCHUNK)
        def _():
            _gather(nxt, buf)
        return _

    lax.fori_loop(0, NCHUNK, body, None)

    pltpu.sync_copy(acc, out_hbm.at[pl.ds(wid * B_PER_W * C8,
                                          B_PER_W * C8)])


def _sc_pool(x_chunks, pat, binit, planes):
    mesh = plsc.VectorSubcoreMesh(core_axis_name="c", subcore_axis_name="s")
    kern = pl.kernel(
        _sc_pool_body,
        out_type=jax.ShapeDtypeStruct((B * C8,), jnp.float32),
        mesh=mesh,
        scratch_types=[
            pltpu.VMEM((NCHUNK, CHUNK), jnp.int32),
            pltpu.VMEM((NCLS * VPC, LANES), jnp.int32),
            pltpu.VMEM((NBUF * NP * CHUNK,), jnp.float32),
            pltpu.VMEM((B_PER_W * C8,), jnp.float32),
            pltpu.SemaphoreType.DMA,
        ],
        compiler_params=pltpu.CompilerParams(use_tc_tiling_on_sc=False,
                                             needs_layout_passes=False),
    )
    return kern(x_chunks, pat, binit, *planes)


def kernel(x, table, W, b):
    table_t = table.T
    wp = jnp.zeros((C8, D), jnp.float32).at[:NCLS].set(W)
    planes = _tw_planes(wp, table_t)

    x_chunks = x.astype(jnp.int32).reshape(NW, B_PER_W, L).transpose(0, 2, 1)
    pat = (jnp.arange(NCLS * VPC, dtype=jnp.int32)[:, None] % VPC * (LANES * C8)
           + jnp.arange(LANES, dtype=jnp.int32)[None, :] * C8
           + jnp.arange(NCLS * VPC, dtype=jnp.int32)[:, None] // VPC)
    binit = jnp.broadcast_to(
        jnp.concatenate([b, jnp.zeros((C8 - NCLS,), jnp.float32)]),
        (B_PER_W, C8)).reshape(B_PER_W * C8)
    raw = _sc_pool(x_chunks, pat, binit, planes)
    return raw.reshape(B, C8)[:, :NCLS]


# speedup vs baseline: 1.0118x; 1.0118x over previous
import jax
import jax.numpy as jnp
from jax import lax
from jax.experimental import pallas as pl
from jax.experimental.pallas import tpu as pltpu
from jax.experimental.pallas import tpu_sc as plsc

B = 4096
L = 50
D = 64
NCLS = 6
V = 1000000
C8 = 8

NC = 2
NS = 16
NW = NC * NS
B_PER_W = B // NW
CHUNK = 128
NCHUNK = (B_PER_W * L) // CHUNK
ROWS_PER_SC = B // NC

BN = 32768


def _tw_body(wp_ref, tt_ref, *plane_refs):
    res = lax.dot_general(
        wp_ref[...].astype(jnp.bfloat16), tt_ref[...].astype(jnp.bfloat16),
        dimension_numbers=(((1,), (0,)), ((), ())),
        preferred_element_type=jnp.float32,
    ) * (1.0 / L)
    for p in range(NP):
        lo = lax.bitcast_convert_type(
            res[2 * p, :].astype(jnp.bfloat16), jnp.uint16).astype(jnp.uint32)
        hi = lax.bitcast_convert_type(
            res[2 * p + 1, :].astype(jnp.bfloat16), jnp.uint16).astype(jnp.uint32)
        plane_refs[p][...] = lax.bitcast_convert_type(
            (hi << 16) | lo, jnp.float32)


def _tw_planes(wp, table_t):
    return pl.pallas_call(
        _tw_body,
        grid=(pl.cdiv(V, BN),),
        in_specs=[
            pl.BlockSpec((C8, D), lambda i: (0, 0)),
            pl.BlockSpec((D, BN), lambda i: (0, i)),
        ],
        out_specs=[pl.BlockSpec((BN,), lambda i: (i,))] * NP,
        out_shape=[jax.ShapeDtypeStruct((V,), jnp.float32)] * NP,
    )(wp, table_t)


NP = NCLS // 2
NBUF = 8
LANES = 16
VPC = CHUNK // LANES


def _sc_pool_body(x_hbm, pat_hbm, binit_hbm,
                  p0, p1, p2,
                  out_hbm, idxs, pat, rows, acc, gsem):
    planes = (p0, p1, p2)
    c = lax.axis_index("c")
    s = lax.axis_index("s")
    wid = c * NS + s

    pltpu.sync_copy(x_hbm.at[wid], idxs)
    pltpu.sync_copy(pat_hbm, pat)
    pltpu.sync_copy(binit_hbm, acc)

    def _gather(g, buf):
        for k in range(NP):
            pltpu.async_copy(planes[k].at[idxs.at[g]],
                             rows.at[pl.ds((buf * NP + k) * CHUNK, CHUNK)],
                             gsem)

    def _wait(g, buf):
        for k in range(NP):
            pltpu.make_async_copy(
                planes[k].at[idxs.at[g]],
                rows.at[pl.ds((buf * NP + k) * CHUNK, CHUNK)],
                gsem).wait()

    for g in range(NBUF):
        _gather(g, g)

    def body(g, _):
        buf = lax.rem(g, NBUF)
        _wait(g, buf)
        for k in range(NP):
            for j in range(VPC):
                val = rows[pl.ds((buf * NP + k) * CHUNK + j * LANES,
                                 LANES)]
                u = lax.bitcast_convert_type(val, jnp.uint32)
                lo = lax.bitcast_convert_type(u << 16, jnp.float32)
                hi = lax.bitcast_convert_type(
                    u & jnp.uint32(0xFFFF0000), jnp.float32)
                plsc.addupdate_scatter(
                    acc, [pat[(2 * k) * VPC + j, :]], lo)
                plsc.addupdate_scatter(
                    acc, [pat[(2 * k + 1) * VPC + j, :]], hi)
        nxt = g + NBUF

        @pl.when(nxt < NCHUNK)
        def _():
            _gather(nxt, buf)
        return _

    lax.fori_loop(0, NCHUNK, body, None)

    pltpu.sync_copy(acc, out_hbm.at[pl.ds(wid * B_PER_W * C8,
                                          B_PER_W * C8)])


def _sc_pool(x_chunks, pat, binit, planes):
    mesh = plsc.VectorSubcoreMesh(core_axis_name="c", subcore_axis_name="s")
    kern = pl.kernel(
        _sc_pool_body,
        out_type=jax.ShapeDtypeStruct((B * C8,), jnp.float32),
        mesh=mesh,
        scratch_types=[
            pltpu.VMEM((NCHUNK, CHUNK), jnp.int32),
            pltpu.VMEM((NCLS * VPC, LANES), jnp.int32),
            pltpu.VMEM((NBUF * NP * CHUNK,), jnp.float32),
            pltpu.VMEM((B_PER_W * C8,), jnp.float32),
            pltpu.SemaphoreType.DMA,
        ],
        compiler_params=pltpu.CompilerParams(use_tc_tiling_on_sc=False,
                                             needs_layout_passes=False),
    )
    return kern(x_chunks, pat, binit, *planes)


def kernel(x, table, W, b):
    table_t = table.T
    wp = jnp.zeros((C8, D), jnp.float32).at[:NCLS].set(W)
    planes = _tw_planes(wp, table_t)

    x_chunks = x.astype(jnp.int32).reshape(NW, B_PER_W, L).transpose(0, 2, 1)
    pat = (jnp.arange(NCLS * VPC, dtype=jnp.int32)[:, None] % VPC * (LANES * C8)
           + jnp.arange(LANES, dtype=jnp.int32)[None, :] * C8
           + jnp.arange(NCLS * VPC, dtype=jnp.int32)[:, None] // VPC)
    binit = jnp.broadcast_to(
        jnp.concatenate([b, jnp.zeros((C8 - NCLS,), jnp.float32)]),
        (B_PER_W, C8)).reshape(B_PER_W * C8)
    raw = _sc_pool(x_chunks, pat, binit, planes)
    return raw.reshape(B, C8)[:, :NCLS]
